# baseline (speedup 1.0000x reference)
import functools

import jax
import jax.numpy as jnp
from jax import lax
from jax.experimental import pallas as pl
from jax.experimental.pallas import tpu as pltpu
from jax.experimental.pallas import tpu_sc as plsc

VOCAB = 100000
DIM = 1024
B, S = 4, 4096
TOKENS = B * S
EPS = 1e-12

NC, NS, L = 2, 16, 16
NW = NC * NS
TOK_PER_W = TOKENS // NW
CHUNK = 32
NCHUNK = TOK_PER_W // CHUNK
NSL = DIM // L

_mesh = plsc.VectorSubcoreMesh(
    core_axis_name="c", subcore_axis_name="s", num_cores=NC, num_subcores=NS
)


@functools.partial(
    pl.kernel,
    out_type=jax.ShapeDtypeStruct((TOKENS, DIM), jnp.float32),
    mesh=_mesh,
    compiler_params=pltpu.CompilerParams(needs_layout_passes=False),
    scratch_types=dict(
        idx_v=pltpu.VMEM((CHUNK,), jnp.int32),
        msk_v=pltpu.VMEM((CHUNK,), jnp.int32),
        t2_v=pltpu.VMEM((2, DIM), jnp.float32),
        p2_v=pltpu.VMEM((2, DIM), jnp.float32),
        base_v=pltpu.VMEM((DIM,), jnp.float32),
        delta_v=pltpu.VMEM((DIM,), jnp.float32),
        g_v=pltpu.VMEM((DIM,), jnp.float32),
        b_v=pltpu.VMEM((DIM,), jnp.float32),
        x_v=pltpu.VMEM((CHUNK, DIM), jnp.float32),
        sem=pltpu.SemaphoreType.DMA,
    ),
)
def _emb_ln_kernel(ids_hbm, msk_hbm, word_hbm, pos_hbm, type_hbm, gamma_hbm,
                   beta_hbm, out_hbm, *, idx_v, msk_v, t2_v, p2_v, base_v,
                   delta_v, g_v, b_v, x_v, sem):
    wid = lax.axis_index("s") * NC + lax.axis_index("c")

    pltpu.sync_copy(type_hbm, t2_v)
    pltpu.sync_copy(pos_hbm.at[pl.ds(0, 2)], p2_v)
    pltpu.sync_copy(gamma_hbm, g_v)
    pltpu.sync_copy(beta_hbm, b_v)
    for j in range(NSL):
        sl = pl.ds(j * L, L)
        c0 = t2_v[0, sl] + p2_v[0, sl]
        c1 = t2_v[1, sl] + p2_v[1, sl]
        base_v[sl] = c0
        delta_v[sl] = c1 - c0

    def token_body(t, _):
        t16 = jnp.full((L,), t, dtype=jnp.int32)
        mf = plsc.load_gather(msk_v, [t16]).astype(jnp.float32)

        sx = jnp.zeros((L,), jnp.float32)
        sq = jnp.zeros((L,), jnp.float32)
        for j in range(NSL):
            sl = pl.ds(j * L, L)
            v = x_v[t, sl] + base_v[sl] + mf * delta_v[sl]
            x_v[t, sl] = v
            sx = sx + v
            sq = sq + v * v
        s1 = jnp.sum(sx)
        s2 = jnp.sum(sq)
        mean = s1 * (1.0 / DIM)
        var = s2 * (1.0 / DIM) - mean * mean

        a = jnp.full((L,), var + EPS, jnp.float32)
        i = plsc.bitcast(a, jnp.int32)
        y = plsc.bitcast(jnp.int32(0x5F3759DF) - (i >> 1), jnp.float32)
        half = -0.5 * a
        for _ in range(3):
            y = y * (1.5 + half * y * y)

        p = y
        q = jnp.full((L,), -mean, jnp.float32) * y

        for j in range(NSL):
            sl = pl.ds(j * L, L)
            v = x_v[t, sl]
            x_v[t, sl] = (v * p + q) * g_v[sl] + b_v[sl]
        return 0

    def chunk_body(c, _):
        tok = wid * TOK_PER_W + c * CHUNK
        rows = pl.ds(tok, CHUNK)
        pltpu.sync_copy(ids_hbm.at[rows], idx_v)
        pltpu.sync_copy(msk_hbm.at[rows], msk_v)
        pltpu.async_copy(word_hbm.at[idx_v], x_v, sem).wait()
        lax.fori_loop(0, CHUNK, token_body, 0)
        pltpu.sync_copy(x_v, out_hbm.at[rows])
        return 0

    lax.fori_loop(0, NCHUNK, chunk_body, 0)


def kernel(input_ids, attention_mask, token_type_ids, word_emb, pos_emb,
           type_emb, ln_gamma, ln_beta):
    del token_type_ids
    ids = input_ids.reshape(-1).astype(jnp.int32)
    msk = attention_mask.reshape(-1).astype(jnp.int32)
    out = _emb_ln_kernel(ids, msk, word_emb, pos_emb, type_emb,
                         ln_gamma, ln_beta)
    return out.reshape(B, S, DIM)
